# baseline (speedup 1.0000x reference)
import jax
import jax.numpy as jnp
from jax.experimental import pallas as pl


def _body(row_ref, col_ref, out_ref):
    h = row_ref.shape[0]
    w = col_ref.shape[0]
    nf = row_ref.shape[1]
    colT = col_ref[...].T
    rowT = row_ref[...].T
    top = jnp.broadcast_to(colT[:, None, :], (nf, h, w)).reshape(nf, h * w)
    bot = jnp.broadcast_to(rowT[:, :, None], (nf, h, w)).reshape(nf, h * w)
    out_ref[0] = jnp.concatenate([top, bot], axis=0)


def kernel(mask, row_embed, col_embed):
    bs = mask.shape[0]
    h, w = mask.shape[-2:]
    nf = row_embed.shape[1]
    row = row_embed[:h]
    col = col_embed[:w]
    out = pl.pallas_call(
        _body,
        grid=(bs,),
        in_specs=[
            pl.BlockSpec((h, nf), lambda b: (0, 0)),
            pl.BlockSpec((w, nf), lambda b: (0, 0)),
        ],
        out_specs=pl.BlockSpec((1, 2 * nf, h * w), lambda b: (b, 0, 0)),
        out_shape=jax.ShapeDtypeStruct((bs, 2 * nf, h * w), jnp.float32),
    )(row, col)
    return out.reshape(bs, 2 * nf, h, w)
